# baseline (speedup 1.0000x reference)
import jax
import jax.numpy as jnp
from jax.experimental import pallas as pl

N = 50000


def _gcn_conv(x, src, dst, W, b, n):
    h = x @ W.T
    deg = jnp.zeros((n,), dtype=h.dtype).at[dst].add(1.0)
    dinv = jnp.where(deg > 0, 1.0 / jnp.sqrt(deg), 0.0)
    norm = dinv[src] * dinv[dst]
    msg = h[src] * norm[:, None]
    out = jnp.zeros((n, h.shape[1]), dtype=h.dtype).at[dst].add(msg)
    return out + b


def _head_body(h_ref, wr_ref, br_ref, o_ref):
    r = h_ref[...]
    w = wr_ref[...]
    o_ref[...] = jax.nn.sigmoid(
        jnp.sum(r * w, axis=1, keepdims=True) + br_ref[0, 0])


def kernel(x, edge_index, W1, b1, W2, b2, W3, b3, Wr, br):
    src = edge_index[0]
    dst = edge_index[1]
    loop = jnp.arange(N, dtype=src.dtype)
    src = jnp.concatenate([src, loop])
    dst = jnp.concatenate([dst, loop])
    h = jax.nn.relu(_gcn_conv(x, src, dst, W1, b1, N))
    h = jax.nn.relu(_gcn_conv(h, src, dst, W2, b2, N))
    h = jax.nn.relu(_gcn_conv(h, src, dst, W3, b3, N))
    bn = 2000
    out = pl.pallas_call(
        _head_body,
        grid=(N // bn,),
        in_specs=[
            pl.BlockSpec((bn, 64), lambda i: (i, 0)),
            pl.BlockSpec((1, 64), lambda i: (0, 0)),
            pl.BlockSpec((1, 1), lambda i: (0, 0)),
        ],
        out_specs=pl.BlockSpec((bn, 1), lambda i: (i, 0)),
        out_shape=jax.ShapeDtypeStruct((N, 1), jnp.float32),
    )(h, Wr, br.reshape(1, 1))
    return out[:, 0]


# speedup vs baseline: 5.6676x; 5.6676x over previous
import functools

import jax
import jax.numpy as jnp
from jax import lax
from jax.experimental import pallas as pl
from jax.experimental.pallas import tpu as pltpu
from jax.experimental.pallas import tpu_sc as plsc

N = 50000
E = 800000
EP = 819200
EPW = EP // 32
NBATCH = EPW // 128
NACC = 51200
DUMMY = N
STRIPE = NACC // 16
ZROWS = 128
CW = 32
SEG = 25
NSEG = NBATCH // SEG
BN = 2000

_mesh = plsc.VectorSubcoreMesh(core_axis_name="c", subcore_axis_name="s")


@functools.partial(
    pl.kernel,
    out_type=jax.ShapeDtypeStruct((32, NACC), jnp.float32),
    mesh=_mesh,
    scratch_types=[
        pltpu.VMEM((EPW,), jnp.int32),
        pltpu.VMEM((NACC,), jnp.float32),
    ],
    compiler_params=pltpu.CompilerParams(needs_layout_passes=False),
)
def _deg_kernel(dst_hbm, out_hbm, dst_v, deg_v):
    c = lax.axis_index("c")
    s = lax.axis_index("s")
    w = s * 2 + c
    pltpu.sync_copy(dst_hbm.at[pl.ds(w * EPW, EPW)], dst_v)

    zero16 = jnp.zeros((16,), jnp.float32)

    def zbody(i, _):
        deg_v[pl.ds(i * 16, 16)] = zero16
        return 0

    lax.fori_loop(0, NACC // 16, zbody, 0)

    ones = jnp.ones((16,), jnp.float32)

    def body(i, _):
        idx = dst_v[pl.ds(i * 16, 16)]
        plsc.addupdate_scatter(deg_v, [idx], ones)
        return 0

    lax.fori_loop(0, EPW // 16, body, 0)
    pltpu.sync_copy(deg_v, out_hbm.at[w])


def _make_scatter_kernel(nchunks):
    def body(*refs):
        tables = refs[:nchunks]
        src_v2, dst_v2 = refs[nchunks], refs[nchunks + 1]
        out_hbm = refs[nchunks + 2]
        src_v, dst_v, gbuf, zbuf, acc, sem = refs[nchunks + 3:]

        c = lax.axis_index("c")
        s = lax.axis_index("s")
        w = s * 2 + c

        zero16 = jnp.zeros((16,), jnp.float32)

        def zb_body(i, _):
            zbuf[i, pl.ds(0, 16)] = zero16
            zbuf[i, pl.ds(16, 16)] = zero16
            return 0

        lax.fori_loop(0, ZROWS, zb_body, 0)

        base = s * STRIPE
        for cidx in range(nchunks):
            table = tables[cidx]
            for k in range(STRIPE // ZROWS):
                pltpu.sync_copy(zbuf, acc.at[pl.ds(base + k * ZROWS, ZROWS), :])
            plsc.subcore_barrier()

            def seg_body(g, _):
                row0 = w * NBATCH + g * SEG
                pltpu.sync_copy(src_v2.at[pl.ds(row0, SEG)], src_v)
                pltpu.sync_copy(dst_v2.at[pl.ds(row0, SEG)], dst_v)

                def sbody(j, _):
                    pltpu.async_copy(table.at[src_v.at[j]], gbuf, sem).wait()
                    pltpu.sync_copy(gbuf, acc.at[dst_v.at[j]], add=True)
                    return 0

                lax.fori_loop(0, SEG, sbody, 0)
                return 0

            lax.fori_loop(0, NSEG, seg_body, 0)
            plsc.subcore_barrier()
            pltpu.sync_copy(
                acc.at[pl.ds(base, STRIPE), :],
                out_hbm.at[c, cidx, pl.ds(base, STRIPE), :],
            )
            plsc.subcore_barrier()

    return pl.kernel(
        body,
        out_type=jax.ShapeDtypeStruct((2, nchunks, NACC, CW), jnp.float32),
        mesh=_mesh,
        scratch_types=[
            pltpu.VMEM((SEG, 128), jnp.int32),
            pltpu.VMEM((SEG, 128), jnp.int32),
            pltpu.VMEM((128, CW), jnp.float32),
            pltpu.VMEM((ZROWS, CW), jnp.float32),
            pltpu.VMEM_SHARED((NACC, CW), jnp.float32),
            pltpu.SemaphoreType.DMA,
        ],
        compiler_params=pltpu.CompilerParams(
            needs_layout_passes=False, use_tc_tiling_on_sc=False),
    )


_scatterA = _make_scatter_kernel(128 // CW)
_scatterB = _make_scatter_kernel(64 // CW)


def _dinv_body(p_ref, o_ref):
    sm = jnp.sum(p_ref[...], axis=0, keepdims=True) + 1.0
    o_ref[...] = lax.rsqrt(sm)


def _l1_body(x_ref, w_ref, dinv_ref, *outs):
    a = jnp.dot(x_ref[...], w_ref[...], preferred_element_type=jnp.float32)
    hs = a * dinv_ref[...]
    for ci in range(len(outs)):
        outs[ci][...] = hs[:, ci * CW:(ci + 1) * CW]


def _mid_body(nc_in, nc_out, p_ref, dinv_ref, w_ref, b_ref, *refs):
    hs_refs = refs[:nc_in]
    outs = refs[nc_in:]
    p = p_ref[...]
    agg = jnp.concatenate(
        [p[0, ci] + p[1, ci] + hs_refs[ci][...] for ci in range(nc_in)], axis=1)
    dinv = dinv_ref[...]
    z = agg * dinv + b_ref[...]
    r = jnp.maximum(z, 0.0)
    a = jnp.dot(r, w_ref[...], preferred_element_type=jnp.float32)
    hs = a * dinv
    for ci in range(nc_out):
        outs[ci][...] = hs[:, ci * CW:(ci + 1) * CW]


def _head_body(p_ref, dinv_ref, b_ref, wr_ref, br_ref, *refs):
    hs_refs = refs[:-1]
    o_ref = refs[-1]
    p = p_ref[...]
    agg = jnp.concatenate(
        [p[0, ci] + p[1, ci] + hs_refs[ci][...] for ci in range(len(hs_refs))],
        axis=1)
    z = agg * dinv_ref[...] + b_ref[...]
    r = jnp.maximum(z, 0.0)
    o_ref[...] = jax.nn.sigmoid(
        jnp.sum(r * wr_ref[...], axis=1, keepdims=True) + br_ref[0, 0])


def _full(shape):
    return pl.BlockSpec(shape, lambda i: tuple(0 for _ in shape))


def kernel(x, edge_index, W1, b1, W2, b2, W3, b3, Wr, br):
    src = edge_index[0]
    dst = edge_index[1]
    src_p = jnp.concatenate([src, jnp.zeros((EP - E,), src.dtype)])
    dst_p = jnp.concatenate([dst, jnp.full((EP - E,), DUMMY, dst.dtype)])
    src2d = src_p.reshape(EP // 128, 128)
    dst2d = dst_p.reshape(EP // 128, 128)

    xp = jnp.pad(x, ((0, 0), (0, 128 - x.shape[1])))
    W1T = jnp.pad(W1.T, ((0, 128 - W1.shape[1]), (0, 0)))
    W2T = W2.T
    W3T = W3.T

    deg_parts = _deg_kernel(dst_p)

    nb = N // BN
    dinv_row = pl.pallas_call(
        _dinv_body,
        grid=(NACC // 2048,),
        in_specs=[pl.BlockSpec((32, 2048), lambda i: (0, i))],
        out_specs=pl.BlockSpec((1, 2048), lambda i: (0, i)),
        out_shape=jax.ShapeDtypeStruct((1, NACC), jnp.float32),
    )(deg_parts)
    dinv_col = dinv_row[0, :N].reshape(N, 1)

    chunk_shape = jax.ShapeDtypeStruct((N, CW), jnp.float32)
    chunk_spec = pl.BlockSpec((BN, CW), lambda i: (i, 0))
    nc1 = 128 // CW
    nc3 = 64 // CW

    hs1 = pl.pallas_call(
        _l1_body,
        grid=(nb,),
        in_specs=[
            pl.BlockSpec((BN, 128), lambda i: (i, 0)),
            _full((128, 128)),
            pl.BlockSpec((BN, 1), lambda i: (i, 0)),
        ],
        out_specs=[chunk_spec] * nc1,
        out_shape=[chunk_shape] * nc1,
    )(xp, W1T, dinv_col)

    p1 = _scatterA(*hs1, src2d, dst2d)

    hs2 = pl.pallas_call(
        functools.partial(_mid_body, nc1, nc1),
        grid=(nb,),
        in_specs=[
            pl.BlockSpec((2, nc1, BN, CW), lambda i: (0, 0, i, 0)),
            pl.BlockSpec((BN, 1), lambda i: (i, 0)),
            _full((128, 128)),
            _full((1, 128)),
        ] + [chunk_spec] * nc1,
        out_specs=[chunk_spec] * nc1,
        out_shape=[chunk_shape] * nc1,
    )(p1, dinv_col, W2T, b1.reshape(1, 128), *hs1)

    p2 = _scatterA(*hs2, src2d, dst2d)

    hs3 = pl.pallas_call(
        functools.partial(_mid_body, nc1, nc3),
        grid=(nb,),
        in_specs=[
            pl.BlockSpec((2, nc1, BN, CW), lambda i: (0, 0, i, 0)),
            pl.BlockSpec((BN, 1), lambda i: (i, 0)),
            _full((128, 64)),
            _full((1, 128)),
        ] + [chunk_spec] * nc1,
        out_specs=[chunk_spec] * nc3,
        out_shape=[chunk_shape] * nc3,
    )(p2, dinv_col, W3T, b2.reshape(1, 128), *hs2)

    p3 = _scatterB(*hs3, src2d, dst2d)

    out = pl.pallas_call(
        _head_body,
        grid=(nb,),
        in_specs=[
            pl.BlockSpec((2, nc3, BN, CW), lambda i: (0, 0, i, 0)),
            pl.BlockSpec((BN, 1), lambda i: (i, 0)),
            _full((1, 64)),
            _full((1, 64)),
            _full((1, 1)),
        ] + [chunk_spec] * nc3,
        out_specs=pl.BlockSpec((BN, 1), lambda i: (i, 0)),
        out_shape=jax.ShapeDtypeStruct((N, 1), jnp.float32),
    )(p3, dinv_col, b3.reshape(1, 64), Wr, br.reshape(1, 1), *hs3)
    return out[:, 0]


# speedup vs baseline: 7.1301x; 1.2580x over previous
import functools

import jax
import jax.numpy as jnp
from jax import lax
from jax.experimental import pallas as pl
from jax.experimental.pallas import tpu as pltpu
from jax.experimental.pallas import tpu_sc as plsc

N = 50000
E = 800000
EP = 819200
EPW = EP // 32
NBATCH = EPW // 128
NACC = 51200
DUMMY = N
STRIPE = NACC // 16
ZROWS = 128
CW = 32
SEG = 20
NSEG = NBATCH // SEG
NBUF = 4
LOOK = 2
BN = 2000

_mesh = plsc.VectorSubcoreMesh(core_axis_name="c", subcore_axis_name="s")


@functools.partial(
    pl.kernel,
    out_type=jax.ShapeDtypeStruct((32, NACC), jnp.float32),
    mesh=_mesh,
    scratch_types=[
        pltpu.VMEM((EPW,), jnp.int32),
        pltpu.VMEM((NACC,), jnp.float32),
    ],
    compiler_params=pltpu.CompilerParams(needs_layout_passes=False),
)
def _deg_kernel(dst_hbm, out_hbm, dst_v, deg_v):
    c = lax.axis_index("c")
    s = lax.axis_index("s")
    w = s * 2 + c
    pltpu.sync_copy(dst_hbm.at[pl.ds(w * EPW, EPW)], dst_v)

    zero16 = jnp.zeros((16,), jnp.float32)

    def zbody(i, _):
        deg_v[pl.ds(i * 16, 16)] = zero16
        return 0

    lax.fori_loop(0, NACC // 16, zbody, 0)

    ones = jnp.ones((16,), jnp.float32)

    def body(i, _):
        idx = dst_v[pl.ds(i * 16, 16)]
        plsc.addupdate_scatter(deg_v, [idx], ones)
        return 0

    lax.fori_loop(0, EPW // 16, body, 0)
    pltpu.sync_copy(deg_v, out_hbm.at[w])


def _make_scatter_kernel(nchunks):
    def body(*refs):
        tables = refs[:nchunks]
        src_v2, dst_v2 = refs[nchunks], refs[nchunks + 1]
        out_hbm = refs[nchunks + 2]
        rest = refs[nchunks + 3:]
        src_v, dst_v = rest[0], rest[1]
        gbufs = rest[2:2 + NBUF]
        zbuf = rest[2 + NBUF]
        acc = rest[3 + NBUF]
        gsems = rest[4 + NBUF:4 + 2 * NBUF]
        ssems = rest[4 + 2 * NBUF:4 + 3 * NBUF]

        c = lax.axis_index("c")
        s = lax.axis_index("s")
        w = s * 2 + c

        zero16 = jnp.zeros((16,), jnp.float32)

        def zb_body(i, _):
            zbuf[i, pl.ds(0, 16)] = zero16
            zbuf[i, pl.ds(16, 16)] = zero16
            return 0

        lax.fori_loop(0, ZROWS, zb_body, 0)

        base = s * STRIPE
        for cidx in range(nchunks):
            table = tables[cidx]
            for k in range(STRIPE // ZROWS):
                pltpu.sync_copy(zbuf, acc.at[pl.ds(base + k * ZROWS, ZROWS), :])
            plsc.subcore_barrier()

            def seg_body(g, _):
                row0 = w * NBATCH + g * SEG
                pltpu.sync_copy(src_v2.at[pl.ds(row0, SEG)], src_v)
                pltpu.sync_copy(dst_v2.at[pl.ds(row0, SEG)], dst_v)
                pend_g = {}
                pend_s = {}
                for j in range(LOOK):
                    pend_g[j % NBUF] = pltpu.async_copy(
                        table.at[src_v.at[j]], gbufs[j % NBUF], gsems[j % NBUF])
                for j in range(SEG):
                    b = j % NBUF
                    jn = j + LOOK
                    if jn < SEG:
                        bn = jn % NBUF
                        if jn >= NBUF:
                            pend_s.pop(bn).wait()
                        pend_g[bn] = pltpu.async_copy(
                            table.at[src_v.at[jn]], gbufs[bn], gsems[bn])
                    pend_g.pop(b).wait()
                    pend_s[b] = pltpu.async_copy(
                        gbufs[b], acc.at[dst_v.at[j]], ssems[b], add=True)
                for b in sorted(pend_s):
                    pend_s.pop(b).wait()
                return 0

            lax.fori_loop(0, NSEG, seg_body, 0)
            plsc.subcore_barrier()
            pltpu.sync_copy(
                acc.at[pl.ds(base, STRIPE), :],
                out_hbm.at[c, cidx, pl.ds(base, STRIPE), :],
            )
            plsc.subcore_barrier()

    return pl.kernel(
        body,
        out_type=jax.ShapeDtypeStruct((2, nchunks, NACC, CW), jnp.float32),
        mesh=_mesh,
        scratch_types=[
            pltpu.VMEM((SEG, 128), jnp.int32),
            pltpu.VMEM((SEG, 128), jnp.int32),
        ] + [pltpu.VMEM((128, CW), jnp.float32)] * NBUF + [
            pltpu.VMEM((ZROWS, CW), jnp.float32),
            pltpu.VMEM_SHARED((NACC, CW), jnp.float32),
        ] + [pltpu.SemaphoreType.DMA] * (2 * NBUF),
        compiler_params=pltpu.CompilerParams(
            needs_layout_passes=False, use_tc_tiling_on_sc=False),
    )


_scatterA = _make_scatter_kernel(128 // CW)
_scatterB = _make_scatter_kernel(64 // CW)


def _dinv_body(p_ref, o_ref):
    sm = jnp.sum(p_ref[...], axis=0, keepdims=True) + 1.0
    o_ref[...] = lax.rsqrt(sm)


def _l1_body(x_ref, w_ref, dinv_ref, *outs):
    a = jnp.dot(x_ref[...], w_ref[...], preferred_element_type=jnp.float32)
    hs = a * dinv_ref[...]
    for ci in range(len(outs)):
        outs[ci][...] = hs[:, ci * CW:(ci + 1) * CW]


def _mid_body(nc_in, nc_out, p_ref, dinv_ref, w_ref, b_ref, *refs):
    hs_refs = refs[:nc_in]
    outs = refs[nc_in:]
    p = p_ref[...]
    agg = jnp.concatenate(
        [p[0, ci] + p[1, ci] + hs_refs[ci][...] for ci in range(nc_in)], axis=1)
    dinv = dinv_ref[...]
    z = agg * dinv + b_ref[...]
    r = jnp.maximum(z, 0.0)
    a = jnp.dot(r, w_ref[...], preferred_element_type=jnp.float32)
    hs = a * dinv
    for ci in range(nc_out):
        outs[ci][...] = hs[:, ci * CW:(ci + 1) * CW]


def _head_body(p_ref, dinv_ref, b_ref, wr_ref, br_ref, *refs):
    hs_refs = refs[:-1]
    o_ref = refs[-1]
    p = p_ref[...]
    agg = jnp.concatenate(
        [p[0, ci] + p[1, ci] + hs_refs[ci][...] for ci in range(len(hs_refs))],
        axis=1)
    z = agg * dinv_ref[...] + b_ref[...]
    r = jnp.maximum(z, 0.0)
    o_ref[...] = jax.nn.sigmoid(
        jnp.sum(r * wr_ref[...], axis=1, keepdims=True) + br_ref[0, 0])


def _full(shape):
    return pl.BlockSpec(shape, lambda i: tuple(0 for _ in shape))


def kernel(x, edge_index, W1, b1, W2, b2, W3, b3, Wr, br):
    src = edge_index[0]
    dst = edge_index[1]
    src_p = jnp.concatenate([src, jnp.zeros((EP - E,), src.dtype)])
    dst_p = jnp.concatenate([dst, jnp.full((EP - E,), DUMMY, dst.dtype)])
    src2d = src_p.reshape(EP // 128, 128)
    dst2d = dst_p.reshape(EP // 128, 128)

    xp = jnp.pad(x, ((0, 0), (0, 128 - x.shape[1])))
    W1T = jnp.pad(W1.T, ((0, 128 - W1.shape[1]), (0, 0)))
    W2T = W2.T
    W3T = W3.T

    deg_parts = _deg_kernel(dst_p)

    nb = N // BN
    dinv_row = pl.pallas_call(
        _dinv_body,
        grid=(NACC // 2048,),
        in_specs=[pl.BlockSpec((32, 2048), lambda i: (0, i))],
        out_specs=pl.BlockSpec((1, 2048), lambda i: (0, i)),
        out_shape=jax.ShapeDtypeStruct((1, NACC), jnp.float32),
    )(deg_parts)
    dinv_col = dinv_row[0, :N].reshape(N, 1)

    chunk_shape = jax.ShapeDtypeStruct((N, CW), jnp.float32)
    chunk_spec = pl.BlockSpec((BN, CW), lambda i: (i, 0))
    nc1 = 128 // CW
    nc3 = 64 // CW

    hs1 = pl.pallas_call(
        _l1_body,
        grid=(nb,),
        in_specs=[
            pl.BlockSpec((BN, 128), lambda i: (i, 0)),
            _full((128, 128)),
            pl.BlockSpec((BN, 1), lambda i: (i, 0)),
        ],
        out_specs=[chunk_spec] * nc1,
        out_shape=[chunk_shape] * nc1,
    )(xp, W1T, dinv_col)

    p1 = _scatterA(*hs1, src2d, dst2d)

    hs2 = pl.pallas_call(
        functools.partial(_mid_body, nc1, nc1),
        grid=(nb,),
        in_specs=[
            pl.BlockSpec((2, nc1, BN, CW), lambda i: (0, 0, i, 0)),
            pl.BlockSpec((BN, 1), lambda i: (i, 0)),
            _full((128, 128)),
            _full((1, 128)),
        ] + [chunk_spec] * nc1,
        out_specs=[chunk_spec] * nc1,
        out_shape=[chunk_shape] * nc1,
    )(p1, dinv_col, W2T, b1.reshape(1, 128), *hs1)

    p2 = _scatterA(*hs2, src2d, dst2d)

    hs3 = pl.pallas_call(
        functools.partial(_mid_body, nc1, nc3),
        grid=(nb,),
        in_specs=[
            pl.BlockSpec((2, nc1, BN, CW), lambda i: (0, 0, i, 0)),
            pl.BlockSpec((BN, 1), lambda i: (i, 0)),
            _full((128, 64)),
            _full((1, 128)),
        ] + [chunk_spec] * nc1,
        out_specs=[chunk_spec] * nc3,
        out_shape=[chunk_shape] * nc3,
    )(p2, dinv_col, W3T, b2.reshape(1, 128), *hs2)

    p3 = _scatterB(*hs3, src2d, dst2d)

    out = pl.pallas_call(
        _head_body,
        grid=(nb,),
        in_specs=[
            pl.BlockSpec((2, nc3, BN, CW), lambda i: (0, 0, i, 0)),
            pl.BlockSpec((BN, 1), lambda i: (i, 0)),
            _full((1, 64)),
            _full((1, 64)),
            _full((1, 1)),
        ] + [chunk_spec] * nc3,
        out_specs=pl.BlockSpec((BN, 1), lambda i: (i, 0)),
        out_shape=jax.ShapeDtypeStruct((N, 1), jnp.float32),
    )(p3, dinv_col, b3.reshape(1, 64), Wr, br.reshape(1, 1), *hs3)
    return out[:, 0]


# speedup vs baseline: 7.1615x; 1.0044x over previous
import functools

import jax
import jax.numpy as jnp
from jax import lax
from jax.experimental import pallas as pl
from jax.experimental.pallas import tpu as pltpu
from jax.experimental.pallas import tpu_sc as plsc

N = 50000
E = 800000
EP = 819200
EPW = EP // 32
NBATCH = EPW // 128
NACC = 51200
DUMMY = N
STRIPE = NACC // 16
ZROWS = 128
CW = 32
SEG = 20
NSEG = NBATCH // SEG
NBUF = 4
LOOK = 2
BN = 2000

_mesh = plsc.VectorSubcoreMesh(core_axis_name="c", subcore_axis_name="s")


@functools.partial(
    pl.kernel,
    out_type=jax.ShapeDtypeStruct((32, NACC), jnp.float32),
    mesh=_mesh,
    scratch_types=[
        pltpu.VMEM((EPW,), jnp.int32),
        pltpu.VMEM((NACC,), jnp.float32),
    ],
    compiler_params=pltpu.CompilerParams(needs_layout_passes=False),
)
def _deg_kernel(dst_hbm, out_hbm, dst_v, deg_v):
    c = lax.axis_index("c")
    s = lax.axis_index("s")
    w = s * 2 + c
    pltpu.sync_copy(dst_hbm.at[pl.ds(w * EPW, EPW)], dst_v)

    zero16 = jnp.zeros((16,), jnp.float32)

    def zbody(i, _):
        deg_v[pl.ds(i * 16, 16)] = zero16
        return 0

    lax.fori_loop(0, NACC // 16, zbody, 0)

    ones = jnp.ones((16,), jnp.float32)

    def body(i, _):
        idx = dst_v[pl.ds(i * 16, 16)]
        plsc.addupdate_scatter(deg_v, [idx], ones)
        return 0

    lax.fori_loop(0, EPW // 16, body, 0)
    pltpu.sync_copy(deg_v, out_hbm.at[w])


def _make_scatter_kernel(nchunks):
    def body(*refs):
        tables = refs[:nchunks]
        src_v2, dst_v2 = refs[nchunks], refs[nchunks + 1]
        out_hbm = refs[nchunks + 2]
        rest = refs[nchunks + 3:]
        src_v, dst_v = rest[0], rest[1]
        gbufs = rest[2:2 + NBUF]
        zbuf = rest[2 + NBUF]
        acc = rest[3 + NBUF]
        gsems = rest[4 + NBUF:4 + 2 * NBUF]
        ssems = rest[4 + 2 * NBUF:4 + 3 * NBUF]

        c = lax.axis_index("c")
        s = lax.axis_index("s")
        w = s * 2 + c

        zero16 = jnp.zeros((16,), jnp.float32)

        def zb_body(i, _):
            zbuf[i, pl.ds(0, 16)] = zero16
            zbuf[i, pl.ds(16, 16)] = zero16
            return 0

        lax.fori_loop(0, ZROWS, zb_body, 0)

        base = s * STRIPE
        for cidx in range(nchunks):
            table = tables[cidx]
            for k in range(STRIPE // ZROWS):
                pltpu.sync_copy(zbuf, acc.at[pl.ds(base + k * ZROWS, ZROWS), :])
            plsc.subcore_barrier()

            def seg_body(g, _):
                row0 = w * NBATCH + g * SEG
                pltpu.sync_copy(src_v2.at[pl.ds(row0, SEG)], src_v)
                pltpu.sync_copy(dst_v2.at[pl.ds(row0, SEG)], dst_v)
                pend_g = {}
                pend_s = {}
                for j in range(LOOK):
                    pend_g[j % NBUF] = pltpu.async_copy(
                        table.at[src_v.at[j]], gbufs[j % NBUF], gsems[j % NBUF])
                for j in range(SEG):
                    b = j % NBUF
                    jn = j + LOOK
                    if jn < SEG:
                        bn = jn % NBUF
                        if jn >= NBUF:
                            pend_s.pop(bn).wait()
                        pend_g[bn] = pltpu.async_copy(
                            table.at[src_v.at[jn]], gbufs[bn], gsems[bn])
                    pend_g.pop(b).wait()
                    pend_s[b] = pltpu.async_copy(
                        gbufs[b], acc.at[dst_v.at[j]], ssems[b], add=True)
                for b in sorted(pend_s):
                    pend_s.pop(b).wait()
                return 0

            lax.fori_loop(0, NSEG, seg_body, 0)
            plsc.subcore_barrier()
            pltpu.sync_copy(
                acc.at[pl.ds(base, STRIPE), :],
                out_hbm.at[c, cidx, pl.ds(base, STRIPE), :],
            )
            plsc.subcore_barrier()

    return pl.kernel(
        body,
        out_type=jax.ShapeDtypeStruct((2, nchunks, NACC, CW), jnp.float32),
        mesh=_mesh,
        scratch_types=[
            pltpu.VMEM((SEG, 128), jnp.int32),
            pltpu.VMEM((SEG, 128), jnp.int32),
        ] + [pltpu.VMEM((128, CW), jnp.float32)] * NBUF + [
            pltpu.VMEM((ZROWS, CW), jnp.float32),
            pltpu.VMEM_SHARED((NACC, CW), jnp.float32),
        ] + [pltpu.SemaphoreType.DMA] * (2 * NBUF),
        compiler_params=pltpu.CompilerParams(
            needs_layout_passes=False, use_tc_tiling_on_sc=False),
    )


_scatterA = _make_scatter_kernel(128 // CW)
_scatterB = _make_scatter_kernel(64 // CW)


def _dinv_body(p_ref, o_ref):
    sm = jnp.sum(p_ref[...], axis=0, keepdims=True) + 1.0
    o_ref[...] = lax.rsqrt(sm)


def _l1_body(x_ref, w_ref, dinv_ref, *outs):
    a = jnp.dot(x_ref[...], w_ref[...], preferred_element_type=jnp.float32)
    hs = a * dinv_ref[...]
    for ci in range(len(outs)):
        outs[ci][...] = hs[:, ci * CW:(ci + 1) * CW]


def _mid_body(nc_in, nc_out, p_ref, dinv_ref, w_ref, b_ref, *refs):
    hs_refs = refs[:nc_in]
    outs = refs[nc_in:]
    p = p_ref[...]
    agg = jnp.concatenate(
        [p[0, ci] + p[1, ci] + hs_refs[ci][...] for ci in range(nc_in)], axis=1)
    dinv = dinv_ref[...]
    z = agg * dinv + b_ref[...]
    r = jnp.maximum(z, 0.0)
    a = jnp.dot(r, w_ref[...], preferred_element_type=jnp.float32)
    hs = a * dinv
    for ci in range(nc_out):
        outs[ci][...] = hs[:, ci * CW:(ci + 1) * CW]


def _head_body(p_ref, dinv_ref, b_ref, wr_ref, br_ref, *refs):
    hs_refs = refs[:-1]
    o_ref = refs[-1]
    p = p_ref[...]
    agg = jnp.concatenate(
        [p[0, ci] + p[1, ci] + hs_refs[ci][...] for ci in range(len(hs_refs))],
        axis=1)
    z = agg * dinv_ref[...] + b_ref[...]
    r = jnp.maximum(z, 0.0)
    o_ref[...] = jax.nn.sigmoid(
        jnp.sum(r * wr_ref[...], axis=1, keepdims=True) + br_ref[0, 0])


def _full(shape):
    return pl.BlockSpec(shape, lambda i: tuple(0 for _ in shape))


def kernel(x, edge_index, W1, b1, W2, b2, W3, b3, Wr, br):
    src = edge_index[0]
    dst = edge_index[1]
    src_p = jnp.concatenate([src, jnp.zeros((EP - E,), src.dtype)])
    pad_dst = DUMMY + jnp.arange(EP - E, dtype=dst.dtype) % (NACC - N)
    dst_p = jnp.concatenate([dst, pad_dst])
    src2d = src_p.reshape(EP // 128, 128)
    dst2d = dst_p.reshape(EP // 128, 128)

    xp = jnp.pad(x, ((0, 0), (0, 128 - x.shape[1])))
    W1T = jnp.pad(W1.T, ((0, 128 - W1.shape[1]), (0, 0)))
    W2T = W2.T
    W3T = W3.T

    deg_parts = _deg_kernel(dst_p)

    nb = N // BN
    dinv_row = pl.pallas_call(
        _dinv_body,
        grid=(NACC // 2048,),
        in_specs=[pl.BlockSpec((32, 2048), lambda i: (0, i))],
        out_specs=pl.BlockSpec((1, 2048), lambda i: (0, i)),
        out_shape=jax.ShapeDtypeStruct((1, NACC), jnp.float32),
    )(deg_parts)
    dinv_col = dinv_row[0, :N].reshape(N, 1)

    chunk_shape = jax.ShapeDtypeStruct((N, CW), jnp.float32)
    chunk_spec = pl.BlockSpec((BN, CW), lambda i: (i, 0))
    nc1 = 128 // CW
    nc3 = 64 // CW

    hs1 = pl.pallas_call(
        _l1_body,
        grid=(nb,),
        in_specs=[
            pl.BlockSpec((BN, 128), lambda i: (i, 0)),
            _full((128, 128)),
            pl.BlockSpec((BN, 1), lambda i: (i, 0)),
        ],
        out_specs=[chunk_spec] * nc1,
        out_shape=[chunk_shape] * nc1,
    )(xp, W1T, dinv_col)

    p1 = _scatterA(*hs1, src2d, dst2d)

    hs2 = pl.pallas_call(
        functools.partial(_mid_body, nc1, nc1),
        grid=(nb,),
        in_specs=[
            pl.BlockSpec((2, nc1, BN, CW), lambda i: (0, 0, i, 0)),
            pl.BlockSpec((BN, 1), lambda i: (i, 0)),
            _full((128, 128)),
            _full((1, 128)),
        ] + [chunk_spec] * nc1,
        out_specs=[chunk_spec] * nc1,
        out_shape=[chunk_shape] * nc1,
    )(p1, dinv_col, W2T, b1.reshape(1, 128), *hs1)

    p2 = _scatterA(*hs2, src2d, dst2d)

    hs3 = pl.pallas_call(
        functools.partial(_mid_body, nc1, nc3),
        grid=(nb,),
        in_specs=[
            pl.BlockSpec((2, nc1, BN, CW), lambda i: (0, 0, i, 0)),
            pl.BlockSpec((BN, 1), lambda i: (i, 0)),
            _full((128, 64)),
            _full((1, 128)),
        ] + [chunk_spec] * nc1,
        out_specs=[chunk_spec] * nc3,
        out_shape=[chunk_shape] * nc3,
    )(p2, dinv_col, W3T, b2.reshape(1, 128), *hs2)

    p3 = _scatterB(*hs3, src2d, dst2d)

    out = pl.pallas_call(
        _head_body,
        grid=(nb,),
        in_specs=[
            pl.BlockSpec((2, nc3, BN, CW), lambda i: (0, 0, i, 0)),
            pl.BlockSpec((BN, 1), lambda i: (i, 0)),
            _full((1, 64)),
            _full((1, 64)),
            _full((1, 1)),
        ] + [chunk_spec] * nc3,
        out_specs=pl.BlockSpec((BN, 1), lambda i: (i, 0)),
        out_shape=jax.ShapeDtypeStruct((N, 1), jnp.float32),
    )(p3, dinv_col, b3.reshape(1, 64), Wr, br.reshape(1, 1), *hs3)
    return out[:, 0]


# speedup vs baseline: 7.6613x; 1.0698x over previous
import functools

import jax
import jax.numpy as jnp
from jax import lax
from jax.experimental import pallas as pl
from jax.experimental.pallas import tpu as pltpu
from jax.experimental.pallas import tpu_sc as plsc

N = 50000
E = 800000
EP = 819200
EPW = EP // 32
NBATCH = EPW // 128
NACC = 51200
DUMMY = N
STRIPE = NACC // 16
ZROWS = 128
CW = 32
SEG = 16
NBUF = 4
LOOK = 2
FAST_C = 0
NB_F = 288
NB_S = 112
BN = 2000

_mesh = plsc.VectorSubcoreMesh(core_axis_name="c", subcore_axis_name="s")


@functools.partial(
    pl.kernel,
    out_type=jax.ShapeDtypeStruct((32, NACC), jnp.float32),
    mesh=_mesh,
    scratch_types=[
        pltpu.VMEM((EPW,), jnp.int32),
        pltpu.VMEM((NACC,), jnp.float32),
    ],
    compiler_params=pltpu.CompilerParams(needs_layout_passes=False),
)
def _deg_kernel(dst_hbm, out_hbm, dst_v, deg_v):
    c = lax.axis_index("c")
    s = lax.axis_index("s")
    w = s * 2 + c
    pltpu.sync_copy(dst_hbm.at[pl.ds(w * EPW, EPW)], dst_v)

    zero16 = jnp.zeros((16,), jnp.float32)

    def zbody(i, _):
        deg_v[pl.ds(i * 16, 16)] = zero16
        return 0

    lax.fori_loop(0, NACC // 16, zbody, 0)

    ones = jnp.ones((16,), jnp.float32)

    def body(i, _):
        idx = dst_v[pl.ds(i * 16, 16)]
        plsc.addupdate_scatter(deg_v, [idx], ones)
        return 0

    lax.fori_loop(0, EPW // 16, body, 0)
    pltpu.sync_copy(deg_v, out_hbm.at[w])


def _make_scatter_kernel(nchunks):
    def body(*refs):
        tables = refs[:nchunks]
        src_v2, dst_v2 = refs[nchunks], refs[nchunks + 1]
        out_hbm = refs[nchunks + 2]
        rest = refs[nchunks + 3:]
        src_v, dst_v = rest[0], rest[1]
        gbufs = rest[2:2 + NBUF]
        zbuf = rest[2 + NBUF]
        acc = rest[3 + NBUF]
        gsems = rest[4 + NBUF:4 + 2 * NBUF]
        ssems = rest[4 + 2 * NBUF:4 + 3 * NBUF]

        c = lax.axis_index("c")
        s = lax.axis_index("s")
        fast = c == FAST_C
        my_nseg = jnp.where(fast, NB_F // SEG, NB_S // SEG)
        my_start = jnp.where(fast, s * NB_F, 16 * NB_F + s * NB_S)

        zero16 = jnp.zeros((16,), jnp.float32)

        def zb_body(i, _):
            zbuf[i, pl.ds(0, 16)] = zero16
            zbuf[i, pl.ds(16, 16)] = zero16
            return 0

        lax.fori_loop(0, ZROWS, zb_body, 0)

        base = s * STRIPE
        for cidx in range(nchunks):
            table = tables[cidx]
            for k in range(STRIPE // ZROWS):
                pltpu.sync_copy(zbuf, acc.at[pl.ds(base + k * ZROWS, ZROWS), :])
            plsc.subcore_barrier()

            def seg_body(g, _):
                row0 = my_start + g * SEG
                pltpu.sync_copy(src_v2.at[pl.ds(row0, SEG)], src_v)
                pltpu.sync_copy(dst_v2.at[pl.ds(row0, SEG)], dst_v)
                pend_g = {}
                pend_s = {}
                for j in range(LOOK):
                    pend_g[j % NBUF] = pltpu.async_copy(
                        table.at[src_v.at[j]], gbufs[j % NBUF], gsems[j % NBUF])
                for j in range(SEG):
                    b = j % NBUF
                    jn = j + LOOK
                    if jn < SEG:
                        bn = jn % NBUF
                        if jn >= NBUF:
                            pend_s.pop(bn).wait()
                        pend_g[bn] = pltpu.async_copy(
                            table.at[src_v.at[jn]], gbufs[bn], gsems[bn])
                    pend_g.pop(b).wait()
                    pend_s[b] = pltpu.async_copy(
                        gbufs[b], acc.at[dst_v.at[j]], ssems[b], add=True)
                for b in sorted(pend_s):
                    pend_s.pop(b).wait()
                return 0

            lax.fori_loop(0, my_nseg, seg_body, 0)
            plsc.subcore_barrier()
            pltpu.sync_copy(
                acc.at[pl.ds(base, STRIPE), :],
                out_hbm.at[c, cidx, pl.ds(base, STRIPE), :],
            )
            plsc.subcore_barrier()

    return pl.kernel(
        body,
        out_type=jax.ShapeDtypeStruct((2, nchunks, NACC, CW), jnp.float32),
        mesh=_mesh,
        scratch_types=[
            pltpu.VMEM((SEG, 128), jnp.int32),
            pltpu.VMEM((SEG, 128), jnp.int32),
        ] + [pltpu.VMEM((128, CW), jnp.float32)] * NBUF + [
            pltpu.VMEM((ZROWS, CW), jnp.float32),
            pltpu.VMEM_SHARED((NACC, CW), jnp.float32),
        ] + [pltpu.SemaphoreType.DMA] * (2 * NBUF),
        compiler_params=pltpu.CompilerParams(
            needs_layout_passes=False, use_tc_tiling_on_sc=False),
    )


_scatterA = _make_scatter_kernel(128 // CW)
_scatterB = _make_scatter_kernel(64 // CW)


def _dinv_body(p_ref, o_ref):
    sm = jnp.sum(p_ref[...], axis=0, keepdims=True) + 1.0
    o_ref[...] = lax.rsqrt(sm)


def _l1_body(x_ref, w_ref, dinv_ref, *outs):
    a = jnp.dot(x_ref[...], w_ref[...], preferred_element_type=jnp.float32)
    hs = a * dinv_ref[...]
    for ci in range(len(outs)):
        outs[ci][...] = hs[:, ci * CW:(ci + 1) * CW]


def _mid_body(nc_in, nc_out, p_ref, dinv_ref, w_ref, b_ref, *refs):
    hs_refs = refs[:nc_in]
    outs = refs[nc_in:]
    p = p_ref[...]
    agg = jnp.concatenate(
        [p[0, ci] + p[1, ci] + hs_refs[ci][...] for ci in range(nc_in)], axis=1)
    dinv = dinv_ref[...]
    z = agg * dinv + b_ref[...]
    r = jnp.maximum(z, 0.0)
    a = jnp.dot(r, w_ref[...], preferred_element_type=jnp.float32)
    hs = a * dinv
    for ci in range(nc_out):
        outs[ci][...] = hs[:, ci * CW:(ci + 1) * CW]


def _head_body(p_ref, dinv_ref, b_ref, wr_ref, br_ref, *refs):
    hs_refs = refs[:-1]
    o_ref = refs[-1]
    p = p_ref[...]
    agg = jnp.concatenate(
        [p[0, ci] + p[1, ci] + hs_refs[ci][...] for ci in range(len(hs_refs))],
        axis=1)
    z = agg * dinv_ref[...] + b_ref[...]
    r = jnp.maximum(z, 0.0)
    o_ref[...] = jax.nn.sigmoid(
        jnp.sum(r * wr_ref[...], axis=1, keepdims=True) + br_ref[0, 0])


def _full(shape):
    return pl.BlockSpec(shape, lambda i: tuple(0 for _ in shape))


def kernel(x, edge_index, W1, b1, W2, b2, W3, b3, Wr, br):
    src = edge_index[0]
    dst = edge_index[1]
    src_p = jnp.concatenate([src, jnp.zeros((EP - E,), src.dtype)])
    pad_dst = DUMMY + jnp.arange(EP - E, dtype=dst.dtype) % (NACC - N)
    dst_p = jnp.concatenate([dst, pad_dst])
    src2d = src_p.reshape(EP // 128, 128)
    dst2d = dst_p.reshape(EP // 128, 128)

    xp = jnp.pad(x, ((0, 0), (0, 128 - x.shape[1])))
    W1T = jnp.pad(W1.T, ((0, 128 - W1.shape[1]), (0, 0)))
    W2T = W2.T
    W3T = W3.T

    deg_parts = _deg_kernel(dst_p)

    nb = N // BN
    dinv_row = pl.pallas_call(
        _dinv_body,
        grid=(NACC // 2048,),
        in_specs=[pl.BlockSpec((32, 2048), lambda i: (0, i))],
        out_specs=pl.BlockSpec((1, 2048), lambda i: (0, i)),
        out_shape=jax.ShapeDtypeStruct((1, NACC), jnp.float32),
    )(deg_parts)
    dinv_col = dinv_row[0, :N].reshape(N, 1)

    chunk_shape = jax.ShapeDtypeStruct((N, CW), jnp.float32)
    chunk_spec = pl.BlockSpec((BN, CW), lambda i: (i, 0))
    nc1 = 128 // CW
    nc3 = 64 // CW

    hs1 = pl.pallas_call(
        _l1_body,
        grid=(nb,),
        in_specs=[
            pl.BlockSpec((BN, 128), lambda i: (i, 0)),
            _full((128, 128)),
            pl.BlockSpec((BN, 1), lambda i: (i, 0)),
        ],
        out_specs=[chunk_spec] * nc1,
        out_shape=[chunk_shape] * nc1,
    )(xp, W1T, dinv_col)

    p1 = _scatterA(*hs1, src2d, dst2d)

    hs2 = pl.pallas_call(
        functools.partial(_mid_body, nc1, nc1),
        grid=(nb,),
        in_specs=[
            pl.BlockSpec((2, nc1, BN, CW), lambda i: (0, 0, i, 0)),
            pl.BlockSpec((BN, 1), lambda i: (i, 0)),
            _full((128, 128)),
            _full((1, 128)),
        ] + [chunk_spec] * nc1,
        out_specs=[chunk_spec] * nc1,
        out_shape=[chunk_shape] * nc1,
    )(p1, dinv_col, W2T, b1.reshape(1, 128), *hs1)

    p2 = _scatterA(*hs2, src2d, dst2d)

    hs3 = pl.pallas_call(
        functools.partial(_mid_body, nc1, nc3),
        grid=(nb,),
        in_specs=[
            pl.BlockSpec((2, nc1, BN, CW), lambda i: (0, 0, i, 0)),
            pl.BlockSpec((BN, 1), lambda i: (i, 0)),
            _full((128, 64)),
            _full((1, 128)),
        ] + [chunk_spec] * nc1,
        out_specs=[chunk_spec] * nc3,
        out_shape=[chunk_shape] * nc3,
    )(p2, dinv_col, W3T, b2.reshape(1, 128), *hs2)

    p3 = _scatterB(*hs3, src2d, dst2d)

    out = pl.pallas_call(
        _head_body,
        grid=(nb,),
        in_specs=[
            pl.BlockSpec((2, nc3, BN, CW), lambda i: (0, 0, i, 0)),
            pl.BlockSpec((BN, 1), lambda i: (i, 0)),
            _full((1, 64)),
            _full((1, 64)),
            _full((1, 1)),
        ] + [chunk_spec] * nc3,
        out_specs=pl.BlockSpec((BN, 1), lambda i: (i, 0)),
        out_shape=jax.ShapeDtypeStruct((N, 1), jnp.float32),
    )(p3, dinv_col, b3.reshape(1, 64), Wr, br.reshape(1, 1), *hs3)
    return out[:, 0]
